# Initial kernel scaffold; baseline (speedup 1.0000x reference)
import jax
import jax.numpy as jnp
from jax.experimental import pallas as pl


def kernel(x, edge_index, drpt, Wl1, bl1, Wr1, Wl2, bl2, Wr2):
    raise NotImplementedError("write your pallas kernel here")


# speedup vs baseline: 2.9227x; 2.9227x over previous
import functools

import jax
import jax.numpy as jnp
from jax import lax
from jax.experimental import pallas as pl
from jax.experimental.pallas import tpu as pltpu
from jax.experimental.pallas import tpu_sc as plsc

N = 10000
D = 128
E = 320000

NP = 10240
NCORES = 2
NSUB = 16
NW = NCORES * NSUB
CHUNK = 128
CPW = 80
HCH = 16
EP = NW * CPW * CHUNK
RPT = NP // NSUB
NZ = RPT // CHUNK


def _mesh():
    return plsc.VectorSubcoreMesh(
        core_axis_name="c", subcore_axis_name="s", num_cores=NCORES, num_subcores=NSUB
    )


def _fill_const(ref, rows, width, val):
    vec = jnp.full((16,), val, jnp.float32)

    def row(i, carry):
        for k in range(width // 16):
            ref[i, pl.ds(k * 16, 16)] = vec
        return carry

    lax.fori_loop(0, rows, row, 0)


def _agg_body(x_hbm, src_hbm, dst_hbm, part, src_v, dst_v, buf, acc, sem):
    c = lax.axis_index("c")
    s = lax.axis_index("s")
    w = c * NSUB + s

    _fill_const(buf, CHUNK, D, 0.0)

    def zero_chunk(j, carry):
        base = s * RPT + j * CHUNK
        pltpu.sync_copy(buf, acc.at[pl.ds(base, CHUNK)])
        return carry

    lax.fori_loop(0, NZ, zero_chunk, 0)

    plsc.subcore_barrier()

    for hb in range(CPW // HCH):
        pltpu.sync_copy(src_hbm.at[pl.ds(w * CPW + hb * HCH, HCH)], src_v)
        pltpu.sync_copy(dst_hbm.at[pl.ds(w * CPW + hb * HCH, HCH)], dst_v)

        def edge_chunk(j, carry):
            pltpu.async_copy(x_hbm.at[src_v.at[j]], buf, sem).wait()
            pltpu.sync_copy(buf, acc.at[dst_v.at[j]], add=True)
            return carry

        lax.fori_loop(0, HCH, edge_chunk, 0)

    plsc.subcore_barrier()

    def out_chunk(j, carry):
        base = s * RPT + j * CHUNK
        pltpu.sync_copy(acc.at[pl.ds(base, CHUNK)], part.at[c, pl.ds(base, CHUNK)])
        return carry

    lax.fori_loop(0, NZ, out_chunk, 0)


def _make_agg():
    return pl.kernel(
        _agg_body,
        out_type=(jax.ShapeDtypeStruct((NCORES, NP, D), jnp.float32),),
        mesh=_mesh(),
        scratch_types=(
            pltpu.VMEM((HCH, CHUNK), jnp.int32),
            pltpu.VMEM((HCH, CHUNK), jnp.int32),
            pltpu.VMEM((CHUNK, D), jnp.float32),
            pltpu.VMEM_SHARED((NP, D), jnp.float32),
            pltpu.SemaphoreType.DMA,
        ),
        name="sage_agg",
    )


def _cnt_body(dst_hbm, cntp, dst_v, ones_v, acc, sem):
    c = lax.axis_index("c")
    s = lax.axis_index("s")
    w = c * NSUB + s

    _fill_const(ones_v, CHUNK, D, 0.0)

    def zero_chunk(j, carry):
        base = s * RPT + j * CHUNK
        pltpu.sync_copy(ones_v, acc.at[pl.ds(base, CHUNK)])
        return carry

    lax.fori_loop(0, NZ, zero_chunk, 0)
    _fill_const(ones_v, CHUNK, D, 1.0)

    plsc.subcore_barrier()

    for hb in range(CPW // HCH):
        pltpu.sync_copy(dst_hbm.at[pl.ds(w * CPW + hb * HCH, HCH)], dst_v)

        def edge_chunk(j, carry):
            pltpu.sync_copy(ones_v, acc.at[dst_v.at[j]], add=True)
            return carry

        lax.fori_loop(0, HCH, edge_chunk, 0)

    plsc.subcore_barrier()

    def out_chunk(j, carry):
        base = s * RPT + j * CHUNK
        pltpu.sync_copy(acc.at[pl.ds(base, CHUNK)], cntp.at[c, pl.ds(base, CHUNK)])
        return carry

    lax.fori_loop(0, NZ, out_chunk, 0)


def _make_cnt():
    return pl.kernel(
        _cnt_body,
        out_type=(jax.ShapeDtypeStruct((NCORES, NP, D), jnp.float32),),
        mesh=_mesh(),
        scratch_types=(
            pltpu.VMEM((HCH, CHUNK), jnp.int32),
            pltpu.VMEM((CHUNK, D), jnp.float32),
            pltpu.VMEM_SHARED((NP, D), jnp.float32),
            pltpu.SemaphoreType.DMA,
        ),
        name="sage_counts",
    )


def _dense_body(relu, p_ref, c_ref, x_ref, wl_ref, wr_ref, b_ref, o_ref):
    psum = p_ref[0] + p_ref[1]
    cnt = (c_ref[0] + c_ref[1])[:, :1]
    mean = psum * (1.0 / jnp.maximum(cnt, 1.0))
    y = jnp.dot(mean, wl_ref[...], preferred_element_type=jnp.float32)
    y = y + jnp.dot(x_ref[...], wr_ref[...], preferred_element_type=jnp.float32)
    y = y + b_ref[...]
    if relu:
        y = jnp.maximum(y, 0.0)
    o_ref[...] = y


def _dense(p, cntp, x, wlT, wrT, b, relu):
    BN = 512
    return pl.pallas_call(
        functools.partial(_dense_body, relu),
        grid=(NP // BN,),
        in_specs=[
            pl.BlockSpec((NCORES, BN, D), lambda i: (0, i, 0)),
            pl.BlockSpec((NCORES, BN, D), lambda i: (0, i, 0)),
            pl.BlockSpec((BN, D), lambda i: (i, 0)),
            pl.BlockSpec((D, D), lambda i: (0, 0)),
            pl.BlockSpec((D, D), lambda i: (0, 0)),
            pl.BlockSpec((1, D), lambda i: (0, 0)),
        ],
        out_specs=pl.BlockSpec((BN, D), lambda i: (i, 0)),
        out_shape=jax.ShapeDtypeStruct((NP, D), jnp.float32),
    )(p, cntp, x, wlT, wrT, b)


def kernel(x, edge_index, drpt, Wl1, bl1, Wr1, Wl2, bl2, Wr2):
    src = edge_index[0].astype(jnp.int32)
    dst = edge_index[1].astype(jnp.int32)
    srcp = jnp.concatenate([src, jnp.zeros((EP - E,), jnp.int32)])
    dstp = jnp.concatenate([dst, jnp.full((EP - E,), N, jnp.int32)])
    srcp = srcp.reshape(EP // CHUNK, CHUNK)
    dstp = dstp.reshape(EP // CHUNK, CHUNK)
    xp = jnp.pad(x, ((0, NP - N), (0, 0)))

    agg = _make_agg()
    cnt = _make_cnt()

    (cntp,) = cnt(dstp)
    (part1,) = agg(xp, srcp, dstp)
    h = _dense(part1, cntp, xp, Wl1.T, Wr1.T, bl1.reshape(1, D), True)
    (part2,) = agg(h, srcp, dstp)
    out = _dense(part2, cntp, h, Wl2.T, Wr2.T, bl2.reshape(1, D), False)
    return out[:N]


# speedup vs baseline: 8.0742x; 2.7626x over previous
import functools

import jax
import jax.numpy as jnp
from jax import lax
from jax.experimental import pallas as pl
from jax.experimental.pallas import tpu as pltpu
from jax.experimental.pallas import tpu_sc as plsc

N = 10000
D = 128
E = 320000

NP = 10240
NCORES = 2
NSUB = 16
NW = NCORES * NSUB
CHUNK = 128
CPW = 80
HCH = 16
EP = NW * CPW * CHUNK
RPT = NP // NSUB
NZ = RPT // CHUNK


def _mesh():
    return plsc.VectorSubcoreMesh(
        core_axis_name="c", subcore_axis_name="s", num_cores=NCORES, num_subcores=NSUB
    )


def _fill_const(ref, rows, width, val):
    vec = jnp.full((16,), val, jnp.float32)

    def row(i, carry):
        for k in range(width // 16):
            ref[i, pl.ds(k * 16, 16)] = vec
        return carry

    lax.fori_loop(0, rows, row, 0)


def _agg_body(x_hbm, sd_hbm, part, sd_v, buf0, buf1, acc, sem0, sem1):
    c = lax.axis_index("c")
    s = lax.axis_index("s")
    w = c * NSUB + s

    _fill_const(buf0, CHUNK, D, 0.0)

    def zero_chunk(j, carry):
        base = s * RPT + j * CHUNK
        pltpu.sync_copy(buf0, acc.at[pl.ds(base, CHUNK)])
        return carry

    lax.fori_loop(0, NZ, zero_chunk, 0)

    plsc.subcore_barrier()

    def slab(t, carry):
        pltpu.sync_copy(sd_hbm.at[pl.ds(w * (2 * CPW) + t * 8, 8)], sd_v)
        for m in (0, 2):
            h0 = pltpu.async_copy(x_hbm.at[sd_v.at[2 * m]], buf0, sem0)
            h1 = pltpu.async_copy(x_hbm.at[sd_v.at[2 * m + 2]], buf1, sem1)
            h0.wait()
            pltpu.sync_copy(buf0, acc.at[sd_v.at[2 * m + 1]], add=True)
            h1.wait()
            pltpu.sync_copy(buf1, acc.at[sd_v.at[2 * m + 3]], add=True)
        return carry

    lax.fori_loop(0, CPW // 4, slab, 0)

    plsc.subcore_barrier()

    pltpu.sync_copy(acc.at[pl.ds(s * RPT, RPT)], part.at[c, pl.ds(s * RPT, RPT)])


def _make_agg():
    return pl.kernel(
        _agg_body,
        out_type=(jax.ShapeDtypeStruct((NCORES, NP, D), jnp.float32),),
        mesh=_mesh(),
        scratch_types=(
            pltpu.VMEM((8, CHUNK), jnp.int32),
            pltpu.VMEM((CHUNK, D), jnp.float32),
            pltpu.VMEM((CHUNK, D), jnp.float32),
            pltpu.VMEM_SHARED((NP, D), jnp.float32),
            pltpu.SemaphoreType.DMA,
            pltpu.SemaphoreType.DMA,
        ),
        name="sage_agg",
    )


def _cnt_body(sd_hbm, cntp, sd_v, ones_v, acc):
    c = lax.axis_index("c")
    s = lax.axis_index("s")
    w = c * NSUB + s

    _fill_const(ones_v, CHUNK, D, 0.0)

    def zero_chunk(j, carry):
        base = s * RPT + j * CHUNK
        pltpu.sync_copy(ones_v, acc.at[pl.ds(base, CHUNK)])
        return carry

    lax.fori_loop(0, NZ, zero_chunk, 0)
    _fill_const(ones_v, CHUNK, D, 1.0)

    plsc.subcore_barrier()

    def slab(t, carry):
        pltpu.sync_copy(sd_hbm.at[pl.ds(w * (2 * CPW) + t * 8, 8)], sd_v)
        for m in range(4):
            pltpu.sync_copy(ones_v, acc.at[sd_v.at[2 * m + 1]], add=True)
        return carry

    lax.fori_loop(0, CPW // 4, slab, 0)

    plsc.subcore_barrier()

    pltpu.sync_copy(acc.at[pl.ds(s * RPT, RPT)], cntp.at[c, pl.ds(s * RPT, RPT)])


def _make_cnt():
    return pl.kernel(
        _cnt_body,
        out_type=(jax.ShapeDtypeStruct((NCORES, NP, D), jnp.float32),),
        mesh=_mesh(),
        scratch_types=(
            pltpu.VMEM((8, CHUNK), jnp.int32),
            pltpu.VMEM((CHUNK, D), jnp.float32),
            pltpu.VMEM_SHARED((NP, D), jnp.float32),
        ),
        name="sage_counts",
    )


def _dense_body(relu, p_ref, c_ref, x_ref, wl_ref, wr_ref, b_ref, o_ref):
    psum = p_ref[0] + p_ref[1]
    cnt = (c_ref[0] + c_ref[1])[:, :1]
    mean = psum * (1.0 / jnp.maximum(cnt, 1.0))
    y = jnp.dot(mean, wl_ref[...], preferred_element_type=jnp.float32)
    y = y + jnp.dot(x_ref[...], wr_ref[...], preferred_element_type=jnp.float32)
    y = y + b_ref[...]
    if relu:
        y = jnp.maximum(y, 0.0)
    o_ref[...] = y


def _dense(p, cntp, x, wlT, wrT, b, relu):
    BN = 512
    return pl.pallas_call(
        functools.partial(_dense_body, relu),
        grid=(NP // BN,),
        in_specs=[
            pl.BlockSpec((NCORES, BN, D), lambda i: (0, i, 0)),
            pl.BlockSpec((NCORES, BN, D), lambda i: (0, i, 0)),
            pl.BlockSpec((BN, D), lambda i: (i, 0)),
            pl.BlockSpec((D, D), lambda i: (0, 0)),
            pl.BlockSpec((D, D), lambda i: (0, 0)),
            pl.BlockSpec((1, D), lambda i: (0, 0)),
        ],
        out_specs=pl.BlockSpec((BN, D), lambda i: (i, 0)),
        out_shape=jax.ShapeDtypeStruct((NP, D), jnp.float32),
    )(p, cntp, x, wlT, wrT, b)


def kernel(x, edge_index, drpt, Wl1, bl1, Wr1, Wl2, bl2, Wr2):
    src = edge_index[0].astype(jnp.int32)
    dst = edge_index[1].astype(jnp.int32)
    pad = jnp.arange(EP - E, dtype=jnp.int32)
    srcp = jnp.concatenate([src, pad % N])
    dstp = jnp.concatenate([dst, N + pad % (NP - N)])
    srcp = srcp.reshape(EP // CHUNK, CHUNK)
    dstp = dstp.reshape(EP // CHUNK, CHUNK)
    sd = jnp.stack([srcp, dstp], axis=1).reshape(2 * EP // CHUNK, CHUNK)
    xp = jnp.pad(x, ((0, NP - N), (0, 0)))

    agg = _make_agg()
    cnt = _make_cnt()

    (cntp,) = cnt(sd)
    (part1,) = agg(xp, sd)
    h = _dense(part1, cntp, xp, Wl1.T, Wr1.T, bl1.reshape(1, D), True)
    (part2,) = agg(h, sd)
    out = _dense(part2, cntp, h, Wl2.T, Wr2.T, bl2.reshape(1, D), False)
    return out[:N]


# speedup vs baseline: 8.1202x; 1.0057x over previous
import functools

import jax
import jax.numpy as jnp
from jax import lax
from jax.experimental import pallas as pl
from jax.experimental.pallas import tpu as pltpu
from jax.experimental.pallas import tpu_sc as plsc

N = 10000
D = 128
E = 320000

NP = 10240
NCORES = 2
NSUB = 16
NW = NCORES * NSUB
CHUNK = 128
CPW = 80
HCH = 16
EP = NW * CPW * CHUNK
RPT = NP // NSUB
NZ = RPT // CHUNK


def _mesh():
    return plsc.VectorSubcoreMesh(
        core_axis_name="c", subcore_axis_name="s", num_cores=NCORES, num_subcores=NSUB
    )


def _fill_const(ref, rows, width, val):
    vec = jnp.full((16,), val, jnp.float32)

    def row(i, carry):
        for k in range(width // 16):
            ref[i, pl.ds(k * 16, 16)] = vec
        return carry

    lax.fori_loop(0, rows, row, 0)


def _agg_body(x_hbm, sd_hbm, part, sd_v, buf0, buf1, acc, sem0, sem1):
    c = lax.axis_index("c")
    s = lax.axis_index("s")
    w = c * NSUB + s

    _fill_const(buf0, CHUNK, D, 0.0)

    def zero_chunk(j, carry):
        base = s * RPT + j * CHUNK
        pltpu.sync_copy(buf0, acc.at[pl.ds(base, CHUNK)])
        return carry

    lax.fori_loop(0, NZ, zero_chunk, 0)

    plsc.subcore_barrier()

    def slab(t, carry):
        pltpu.sync_copy(sd_hbm.at[pl.ds(w * (2 * CPW) + t * 8, 8)], sd_v)
        for m in (0, 2):
            h0 = pltpu.async_copy(x_hbm.at[sd_v.at[2 * m]], buf0, sem0)
            h1 = pltpu.async_copy(x_hbm.at[sd_v.at[2 * m + 2]], buf1, sem1)
            h0.wait()
            pltpu.sync_copy(buf0, acc.at[sd_v.at[2 * m + 1]], add=True)
            h1.wait()
            pltpu.sync_copy(buf1, acc.at[sd_v.at[2 * m + 3]], add=True)
        return carry

    lax.fori_loop(0, CPW // 4, slab, 0)

    plsc.subcore_barrier()

    pltpu.sync_copy(acc.at[pl.ds(s * RPT, RPT)], part.at[c, pl.ds(s * RPT, RPT)])


def _make_agg():
    return pl.kernel(
        _agg_body,
        out_type=(jax.ShapeDtypeStruct((NCORES, NP, D), jnp.float32),),
        mesh=_mesh(),
        scratch_types=(
            pltpu.VMEM((8, CHUNK), jnp.int32),
            pltpu.VMEM((CHUNK, D), jnp.float32),
            pltpu.VMEM((CHUNK, D), jnp.float32),
            pltpu.VMEM_SHARED((NP, D), jnp.float32),
            pltpu.SemaphoreType.DMA,
            pltpu.SemaphoreType.DMA,
        ),
        name="sage_agg",
    )


def _cnt_body(sd_hbm, cntp, sd_v, ones_v, acc):
    c = lax.axis_index("c")
    s = lax.axis_index("s")
    w = c * NSUB + s

    _fill_const(ones_v, CHUNK, D, 0.0)

    def zero_chunk(j, carry):
        base = s * RPT + j * CHUNK
        pltpu.sync_copy(ones_v, acc.at[pl.ds(base, CHUNK)])
        return carry

    lax.fori_loop(0, NZ, zero_chunk, 0)
    _fill_const(ones_v, CHUNK, D, 1.0)

    plsc.subcore_barrier()

    def slab(t, carry):
        pltpu.sync_copy(sd_hbm.at[pl.ds(w * (2 * CPW) + t * 8, 8)], sd_v)
        for m in range(4):
            pltpu.sync_copy(ones_v, acc.at[sd_v.at[2 * m + 1]], add=True)
        return carry

    lax.fori_loop(0, CPW // 4, slab, 0)

    plsc.subcore_barrier()

    pltpu.sync_copy(acc.at[pl.ds(s * RPT, RPT)], cntp.at[c, pl.ds(s * RPT, RPT)])


def _make_cnt():
    return pl.kernel(
        _cnt_body,
        out_type=(jax.ShapeDtypeStruct((NCORES, NP, D), jnp.float32),),
        mesh=_mesh(),
        scratch_types=(
            pltpu.VMEM((8, CHUNK), jnp.int32),
            pltpu.VMEM((CHUNK, D), jnp.float32),
            pltpu.VMEM_SHARED((NP, D), jnp.float32),
        ),
        name="sage_counts",
    )


def _dense_body(relu, p_ref, c_ref, x_ref, wl_ref, wr_ref, b_ref, o_ref):
    psum = p_ref[0] + p_ref[1]
    cnt = (c_ref[0] + c_ref[1])[:, :1]
    mean = psum * (1.0 / jnp.maximum(cnt, 1.0))
    y = jnp.dot(mean, wl_ref[...], preferred_element_type=jnp.float32)
    y = y + jnp.dot(x_ref[...], wr_ref[...], preferred_element_type=jnp.float32)
    y = y + b_ref[...]
    if relu:
        y = jnp.maximum(y, 0.0)
    o_ref[...] = y


def _dense(p, cntp, x, wlT, wrT, b, relu):
    BN = 400
    return pl.pallas_call(
        functools.partial(_dense_body, relu),
        grid=(N // BN,),
        in_specs=[
            pl.BlockSpec((NCORES, BN, D), lambda i: (0, i, 0)),
            pl.BlockSpec((NCORES, BN, D), lambda i: (0, i, 0)),
            pl.BlockSpec((BN, D), lambda i: (i, 0)),
            pl.BlockSpec((D, D), lambda i: (0, 0)),
            pl.BlockSpec((D, D), lambda i: (0, 0)),
            pl.BlockSpec((1, D), lambda i: (0, 0)),
        ],
        out_specs=pl.BlockSpec((BN, D), lambda i: (i, 0)),
        out_shape=jax.ShapeDtypeStruct((N, D), jnp.float32),
    )(p, cntp, x, wlT, wrT, b)


def kernel(x, edge_index, drpt, Wl1, bl1, Wr1, Wl2, bl2, Wr2):
    src = edge_index[0].astype(jnp.int32)
    dst = edge_index[1].astype(jnp.int32)
    pad = jnp.arange(EP - E, dtype=jnp.int32)
    srcp = jnp.concatenate([src, pad % N])
    dstp = jnp.concatenate([dst, N + pad % (NP - N)])
    srcp = srcp.reshape(EP // CHUNK, CHUNK)
    dstp = dstp.reshape(EP // CHUNK, CHUNK)
    sd = jnp.stack([srcp, dstp], axis=1).reshape(2 * EP // CHUNK, CHUNK)

    agg = _make_agg()
    cnt = _make_cnt()

    (cntp,) = cnt(sd)
    (part1,) = agg(x, sd)
    h = _dense(part1, cntp, x, Wl1.T, Wr1.T, bl1.reshape(1, D), True)
    (part2,) = agg(h, sd)
    out = _dense(part2, cntp, h, Wl2.T, Wr2.T, bl2.reshape(1, D), False)
    return out


# speedup vs baseline: 8.1666x; 1.0057x over previous
import functools

import jax
import jax.numpy as jnp
from jax import lax
from jax.experimental import pallas as pl
from jax.experimental.pallas import tpu as pltpu
from jax.experimental.pallas import tpu_sc as plsc

N = 10000
D = 128
E = 320000

NP = 10240
NCORES = 2
NSUB = 16
NW = NCORES * NSUB
CHUNK = 128
CPW = 80
HCH = 16
EP = NW * CPW * CHUNK
RPT = NP // NSUB
NZ = RPT // CHUNK


def _mesh():
    return plsc.VectorSubcoreMesh(
        core_axis_name="c", subcore_axis_name="s", num_cores=NCORES, num_subcores=NSUB
    )


def _fill_const(ref, rows, width, val):
    vec = jnp.full((16,), val, jnp.float32)

    def row(i, carry):
        for k in range(width // 16):
            ref[i, pl.ds(k * 16, 16)] = vec
        return carry

    lax.fori_loop(0, rows, row, 0)


def _zero_acc(buf0, acc, s):
    def zero_chunk(j, carry):
        base = s * RPT + j * CHUNK
        pltpu.sync_copy(buf0, acc.at[pl.ds(base, CHUNK)])
        return carry

    lax.fori_loop(0, NZ, zero_chunk, 0)


def _agg_phase(x_hbm, sd_hbm, part, sd_v, buf0, buf1, acc, sem0, sem1, c, s, w):
    _fill_const(buf0, CHUNK, D, 0.0)
    _zero_acc(buf0, acc, s)
    plsc.subcore_barrier()

    def slab(t, carry):
        pltpu.sync_copy(sd_hbm.at[pl.ds(w * (2 * CPW) + t * 8, 8)], sd_v)
        for m in (0, 2):
            h0 = pltpu.async_copy(x_hbm.at[sd_v.at[2 * m]], buf0, sem0)
            h1 = pltpu.async_copy(x_hbm.at[sd_v.at[2 * m + 2]], buf1, sem1)
            h0.wait()
            pltpu.sync_copy(buf0, acc.at[sd_v.at[2 * m + 1]], add=True)
            h1.wait()
            pltpu.sync_copy(buf1, acc.at[sd_v.at[2 * m + 3]], add=True)
        return carry

    lax.fori_loop(0, CPW // 4, slab, 0)

    plsc.subcore_barrier()

    pltpu.sync_copy(acc.at[pl.ds(s * RPT, RPT)], part.at[c, pl.ds(s * RPT, RPT)])


def _agg_body(x_hbm, sd_hbm, part, sd_v, buf0, buf1, acc, sem0, sem1):
    c = lax.axis_index("c")
    s = lax.axis_index("s")
    w = c * NSUB + s
    _agg_phase(x_hbm, sd_hbm, part, sd_v, buf0, buf1, acc, sem0, sem1, c, s, w)


def _agg_cnt_body(x_hbm, sd_hbm, part, cntp, sd_v, buf0, buf1, acc, sem0, sem1):
    c = lax.axis_index("c")
    s = lax.axis_index("s")
    w = c * NSUB + s

    _fill_const(buf0, CHUNK, D, 0.0)
    _zero_acc(buf0, acc, s)
    _fill_const(buf0, CHUNK, D, 1.0)
    plsc.subcore_barrier()

    def cnt_slab(t, carry):
        pltpu.sync_copy(sd_hbm.at[pl.ds(w * (2 * CPW) + t * 8, 8)], sd_v)
        for m in range(4):
            pltpu.sync_copy(buf0, acc.at[sd_v.at[2 * m + 1]], add=True)
        return carry

    lax.fori_loop(0, CPW // 4, cnt_slab, 0)
    plsc.subcore_barrier()
    pltpu.sync_copy(acc.at[pl.ds(s * RPT, RPT)], cntp.at[c, pl.ds(s * RPT, RPT)])

    _agg_phase(x_hbm, sd_hbm, part, sd_v, buf0, buf1, acc, sem0, sem1, c, s, w)


def _make_agg(with_counts):
    out_type = [jax.ShapeDtypeStruct((NCORES, NP, D), jnp.float32)]
    if with_counts:
        out_type.append(jax.ShapeDtypeStruct((NCORES, NP, D), jnp.float32))
    return pl.kernel(
        _agg_cnt_body if with_counts else _agg_body,
        out_type=tuple(out_type),
        mesh=_mesh(),
        scratch_types=(
            pltpu.VMEM((8, CHUNK), jnp.int32),
            pltpu.VMEM((CHUNK, D), jnp.float32),
            pltpu.VMEM((CHUNK, D), jnp.float32),
            pltpu.VMEM_SHARED((NP, D), jnp.float32),
            pltpu.SemaphoreType.DMA,
            pltpu.SemaphoreType.DMA,
        ),
        name="sage_agg_cnt" if with_counts else "sage_agg",
    )


def _dense_body(relu, p_ref, c_ref, x_ref, wl_ref, wr_ref, b_ref, o_ref):
    psum = p_ref[0] + p_ref[1]
    cnt = (c_ref[0] + c_ref[1])[:, :1]
    mean = psum * (1.0 / jnp.maximum(cnt, 1.0))
    y = jnp.dot(mean, wl_ref[...], preferred_element_type=jnp.float32)
    y = y + jnp.dot(x_ref[...], wr_ref[...], preferred_element_type=jnp.float32)
    y = y + b_ref[...]
    if relu:
        y = jnp.maximum(y, 0.0)
    o_ref[...] = y


def _dense(p, cntp, x, wlT, wrT, b, relu):
    BN = 400
    return pl.pallas_call(
        functools.partial(_dense_body, relu),
        grid=(N // BN,),
        in_specs=[
            pl.BlockSpec((NCORES, BN, D), lambda i: (0, i, 0)),
            pl.BlockSpec((NCORES, BN, D), lambda i: (0, i, 0)),
            pl.BlockSpec((BN, D), lambda i: (i, 0)),
            pl.BlockSpec((D, D), lambda i: (0, 0)),
            pl.BlockSpec((D, D), lambda i: (0, 0)),
            pl.BlockSpec((1, D), lambda i: (0, 0)),
        ],
        out_specs=pl.BlockSpec((BN, D), lambda i: (i, 0)),
        out_shape=jax.ShapeDtypeStruct((N, D), jnp.float32),
    )(p, cntp, x, wlT, wrT, b)


def kernel(x, edge_index, drpt, Wl1, bl1, Wr1, Wl2, bl2, Wr2):
    src = edge_index[0].astype(jnp.int32)
    dst = edge_index[1].astype(jnp.int32)
    pad = jnp.arange(EP - E, dtype=jnp.int32)
    srcp = jnp.concatenate([src, pad % N])
    dstp = jnp.concatenate([dst, N + pad % (NP - N)])
    srcp = srcp.reshape(EP // CHUNK, CHUNK)
    dstp = dstp.reshape(EP // CHUNK, CHUNK)
    sd = jnp.stack([srcp, dstp], axis=1).reshape(2 * EP // CHUNK, CHUNK)

    agg1 = _make_agg(True)
    agg2 = _make_agg(False)

    part1, cntp = agg1(x, sd)
    h = _dense(part1, cntp, x, Wl1.T, Wr1.T, bl1.reshape(1, D), True)
    (part2,) = agg2(h, sd)
    out = _dense(part2, cntp, h, Wl2.T, Wr2.T, bl2.reshape(1, D), False)
    return out


# speedup vs baseline: 8.1736x; 1.0009x over previous
import functools

import jax
import jax.numpy as jnp
from jax import lax
from jax.experimental import pallas as pl
from jax.experimental.pallas import tpu as pltpu
from jax.experimental.pallas import tpu_sc as plsc

N = 10000
D = 128
E = 320000

NP = 10240
NCORES = 2
NSUB = 16
NW = NCORES * NSUB
CHUNK = 128
CPW = 80
EP = NW * CPW * CHUNK
RPT = NP // NSUB
NZ = RPT // CHUNK


def _mesh():
    return plsc.VectorSubcoreMesh(
        core_axis_name="c", subcore_axis_name="s", num_cores=NCORES, num_subcores=NSUB
    )


def _fill_const(ref, rows, width, val):
    vec = jnp.full((16,), val, jnp.float32)

    def row(i, carry):
        for k in range(width // 16):
            ref[i, pl.ds(k * 16, 16)] = vec
        return carry

    lax.fori_loop(0, rows, row, 0)


def _zero_acc(buf0, acc, s):
    def zero_chunk(j, carry):
        base = s * RPT + j * CHUNK
        pltpu.sync_copy(buf0, acc.at[pl.ds(base, CHUNK)])
        return carry

    lax.fori_loop(0, NZ, zero_chunk, 0)


def _agg_phase(x_hbm, sd_hbm, part, sd_v, buf0, buf1, acc, sem0, sem1, c, s, w):
    _fill_const(buf0, CHUNK, D, 0.0)
    _zero_acc(buf0, acc, s)
    plsc.subcore_barrier()

    def slab(t, carry):
        pltpu.sync_copy(sd_hbm.at[pl.ds(w * (2 * CPW) + t * 8, 8)], sd_v)
        for m in (0, 2):
            h0 = pltpu.async_copy(x_hbm.at[sd_v.at[2 * m]], buf0, sem0)
            h1 = pltpu.async_copy(x_hbm.at[sd_v.at[2 * m + 2]], buf1, sem1)
            h0.wait()
            pltpu.sync_copy(buf0, acc.at[sd_v.at[2 * m + 1]], add=True)
            h1.wait()
            pltpu.sync_copy(buf1, acc.at[sd_v.at[2 * m + 3]], add=True)
        return carry

    lax.fori_loop(0, CPW // 4, slab, 0)

    plsc.subcore_barrier()

    pltpu.sync_copy(acc.at[pl.ds(s * RPT, RPT)], part.at[c, pl.ds(s * RPT, RPT)])


def _agg_body(x_hbm, sd_hbm, part, sd_v, buf0, buf1, acc, sem0, sem1):
    c = lax.axis_index("c")
    s = lax.axis_index("s")
    w = c * NSUB + s
    _agg_phase(x_hbm, sd_hbm, part, sd_v, buf0, buf1, acc, sem0, sem1, c, s, w)


def _agg_cnt_body(x_hbm, sd_hbm, part, cntp, sd_v, buf0, buf1, acc, sem0, sem1):
    c = lax.axis_index("c")
    s = lax.axis_index("s")
    w = c * NSUB + s

    _fill_const(buf0, CHUNK, D, 0.0)
    _zero_acc(buf0, acc, s)
    _fill_const(buf0, CHUNK, D, 1.0)
    plsc.subcore_barrier()

    def cnt_slab(t, carry):
        pltpu.sync_copy(sd_hbm.at[pl.ds(w * (2 * CPW) + t * 8, 8)], sd_v)
        for m in range(4):
            pltpu.sync_copy(buf0, acc.at[sd_v.at[2 * m + 1]], add=True)
        return carry

    lax.fori_loop(0, CPW // 4, cnt_slab, 0)
    plsc.subcore_barrier()
    pltpu.sync_copy(acc.at[pl.ds(s * RPT, RPT)], cntp.at[c, pl.ds(s * RPT, RPT)])

    _agg_phase(x_hbm, sd_hbm, part, sd_v, buf0, buf1, acc, sem0, sem1, c, s, w)


def _make_agg(with_counts):
    out_type = [jax.ShapeDtypeStruct((NCORES, NP, D), jnp.float32)]
    if with_counts:
        out_type.append(jax.ShapeDtypeStruct((NCORES, NP, D), jnp.float32))
    return pl.kernel(
        _agg_cnt_body if with_counts else _agg_body,
        out_type=tuple(out_type),
        mesh=_mesh(),
        scratch_types=(
            pltpu.VMEM((8, CHUNK), jnp.int32),
            pltpu.VMEM((CHUNK, D), jnp.float32),
            pltpu.VMEM((CHUNK, D), jnp.float32),
            pltpu.VMEM_SHARED((NP, D), jnp.float32),
            pltpu.SemaphoreType.DMA,
            pltpu.SemaphoreType.DMA,
        ),
        name="sage_agg_cnt" if with_counts else "sage_agg",
    )


def _dense_body(relu, p_ref, c_ref, x_ref, wl_ref, wr_ref, b_ref, o_ref):
    psum = p_ref[0] + p_ref[1]
    cnt = (c_ref[0] + c_ref[1])[:, :1]
    mean = psum * (1.0 / jnp.maximum(cnt, 1.0))
    y = jnp.dot(mean, wl_ref[...], preferred_element_type=jnp.float32)
    y = y + jnp.dot(x_ref[...], wr_ref[...], preferred_element_type=jnp.float32)
    y = y + b_ref[...]
    if relu:
        y = jnp.maximum(y, 0.0)
    o_ref[...] = y


def _dense(p, cntp, x, wlT, wrT, b, relu):
    BN = 400
    return pl.pallas_call(
        functools.partial(_dense_body, relu),
        grid=(N // BN,),
        in_specs=[
            pl.BlockSpec((NCORES, BN, D), lambda i: (0, i, 0)),
            pl.BlockSpec((NCORES, BN, D), lambda i: (0, i, 0)),
            pl.BlockSpec((BN, D), lambda i: (i, 0)),
            pl.BlockSpec((D, D), lambda i: (0, 0)),
            pl.BlockSpec((D, D), lambda i: (0, 0)),
            pl.BlockSpec((1, D), lambda i: (0, 0)),
        ],
        out_specs=pl.BlockSpec((BN, D), lambda i: (i, 0)),
        out_shape=jax.ShapeDtypeStruct((N, D), jnp.float32),
    )(p, cntp, x, wlT, wrT, b)


def kernel(x, edge_index, drpt, Wl1, bl1, Wr1, Wl2, bl2, Wr2):
    src = edge_index[0].astype(jnp.int32)
    dst = edge_index[1].astype(jnp.int32)
    pad = jnp.arange(EP - E, dtype=jnp.int32)
    srcp = jnp.concatenate([src, pad % N])
    dstp = jnp.concatenate([dst, N + pad % (NP - N)])
    srcp = srcp.reshape(EP // CHUNK, CHUNK)
    dstp = dstp.reshape(EP // CHUNK, CHUNK)
    sd = jnp.stack([srcp, dstp], axis=1).reshape(2 * EP // CHUNK, CHUNK)

    agg1 = _make_agg(True)
    agg2 = _make_agg(False)

    part1, cntp = agg1(x, sd)
    h = _dense(part1, cntp, x, Wl1.T, Wr1.T, bl1.reshape(1, D), True)
    (part2,) = agg2(h, sd)
    out = _dense(part2, cntp, h, Wl2.T, Wr2.T, bl2.reshape(1, D), False)
    return out
